# Initial kernel scaffold; baseline (speedup 1.0000x reference)
import functools

import jax
import jax.numpy as jnp
from jax import lax
from jax.experimental import pallas as pl
from jax.experimental.pallas import tpu as pltpu
from jax.experimental.pallas import tpu_sc as plsc


def _sc_gather(table, idx2d, n_rows, d):
    info = plsc.get_sparse_core_info()
    nc, ns = info.num_cores, info.num_subcores
    nw = nc * ns
    rows_per_w = n_rows // nw
    idxrows_per_w = rows_per_w // 128
    group = 10
    n_groups = idxrows_per_w // group
    buf_rows = group * 128

    mesh = plsc.VectorSubcoreMesh(core_axis_name="c", subcore_axis_name="s")

    @functools.partial(
        pl.kernel,
        mesh=mesh,
        out_type=jax.ShapeDtypeStruct((n_rows, d), jnp.float32),
        scratch_types=[
            pltpu.VMEM((idxrows_per_w, 128), jnp.int32),
            pltpu.VMEM((buf_rows, d), jnp.float32),
            pltpu.SemaphoreType.DMA,
        ],
    )
    def k(table_hbm, idx_hbm, out_hbm, idx_v, rows_v, sem):
        wid = lax.axis_index("s") * nc + lax.axis_index("c")
        pltpu.sync_copy(idx_hbm.at[pl.ds(wid * idxrows_per_w, idxrows_per_w)],
                        idx_v)
        row_base = wid * rows_per_w

        def do_group(g, carry):
            handles = []
            for j in range(group):
                handles.append(pltpu.async_copy(
                    table_hbm.at[idx_v.at[g * group + j]],
                    rows_v.at[pl.ds(j * 128, 128)],
                    sem,
                ))
            for h in handles:
                h.wait()
            pltpu.sync_copy(rows_v,
                            out_hbm.at[pl.ds(row_base + g * buf_rows,
                                             buf_rows)])
            return carry

        lax.fori_loop(0, n_groups, do_group, 0)

    return k(table, idx2d)


def _tc_dense(w_emb3, w_mask, mask_table, Wh0, bh0, Wz0, bz0, Wh1, bh1,
              Wz1, bz1, W_att, b_att, W_emb, b_emb, W_out, b_out):
    B, L, D = w_emb3.shape
    C = W_out.shape[1]
    BB = 256
    grid = (B // BB,)

    def body(x_ref, m_ref, mt_ref, wh0, bh0_, wz0, bz0_, wh1, bh1_, wz1,
             bz1_, watt, batt, wemb, bemb, wout, bout, out_ref):
        gate = jax.nn.sigmoid(mt_ref[...])
        m = m_ref[...]
        x3 = x_ref[...]
        sel = (m < 1)[..., None]
        x3 = x3 * jnp.where(sel, gate[0], gate[1])
        x = x3.reshape(BB * L, D)
        for wh, bh, wz, bz in ((wh0, bh0_, wz0, bz0_),
                               (wh1, bh1_, wz1, bz1_)):
            h = jnp.tanh(
                jnp.dot(x, wh[...], preferred_element_type=jnp.float32)
                + bh[...])
            z = jax.nn.sigmoid(
                jnp.dot(x, wz[...], preferred_element_type=jnp.float32)
                + bz[...])
            x = h * z + x * (1.0 - z)
        att = jax.nn.sigmoid(
            jnp.dot(x, watt[...], preferred_element_type=jnp.float32)
            + batt[...])
        emb = jnp.tanh(
            jnp.dot(x, wemb[...], preferred_element_type=jnp.float32)
            + bemb[...])
        g = (att * emb).reshape(BB, L, D)
        pooled = jnp.mean(g, axis=1) + jnp.max(g, axis=1)
        out_ref[...] = (
            jnp.dot(pooled, wout[...], preferred_element_type=jnp.float32)
            + bout[...])

    full = lambda shape: pl.BlockSpec(shape, lambda i: (0,) * len(shape))
    return pl.pallas_call(
        body,
        grid=grid,
        in_specs=[
            pl.BlockSpec((BB, L, D), lambda i: (i, 0, 0)),
            pl.BlockSpec((BB, L), lambda i: (i, 0)),
            full((2, D)),
            full((D, D)), full((1, D)), full((D, D)), full((1, D)),
            full((D, D)), full((1, D)), full((D, D)), full((1, D)),
            full((D, D)), full((1, D)), full((D, D)), full((1, D)),
            full((D, C)), full((1, C)),
        ],
        out_specs=pl.BlockSpec((BB, C), lambda i: (i, 0)),
        out_shape=jax.ShapeDtypeStruct((B, C), jnp.float32),
    )(w_emb3, w_mask, mask_table,
      Wh0, bh0.reshape(1, D), Wz0, bz0.reshape(1, D),
      Wh1, bh1.reshape(1, D), Wz1, bz1.reshape(1, D),
      W_att, b_att.reshape(1, D), W_emb, b_emb.reshape(1, D),
      W_out, b_out.reshape(1, C))


def kernel(words2ids, w_mask, paris_mat, w_table, mask_table, Wh0, bh0,
           Wz0, bz0, Wh1, bh1, Wz1, bz1, W_att, b_att, W_emb, b_emb,
           W_out, b_out):
    B, L = words2ids.shape
    D = w_table.shape[1]
    n = B * L
    idx2d = words2ids.astype(jnp.int32).reshape(n // 128, 128)
    gathered = _sc_gather(w_table, idx2d, n, D)
    w_emb3 = gathered.reshape(B, L, D)
    return _tc_dense(w_emb3, w_mask.astype(jnp.int32), mask_table,
                     Wh0, bh0, Wz0, bz0, Wh1, bh1, Wz1, bz1,
                     W_att, b_att, W_emb, b_emb, W_out, b_out)


# speedup vs baseline: 1.4166x; 1.4166x over previous
import functools

import jax
import jax.numpy as jnp
from jax import lax
from jax.experimental import pallas as pl
from jax.experimental.pallas import tpu as pltpu
from jax.experimental.pallas import tpu_sc as plsc


def _sc_gather(table, idx2d, n_rows, d):
    info = plsc.get_sparse_core_info()
    nc, ns = info.num_cores, info.num_subcores
    nw = nc * ns
    rows_per_w = n_rows // nw
    idxrows_per_w = rows_per_w // 128
    group = 10
    n_groups = idxrows_per_w // group
    buf_rows = group * 128

    mesh = plsc.VectorSubcoreMesh(core_axis_name="c", subcore_axis_name="s")

    @functools.partial(
        pl.kernel,
        mesh=mesh,
        compiler_params=pltpu.CompilerParams(use_tc_tiling_on_sc=False),
        out_type=jax.ShapeDtypeStruct((n_rows, d), jnp.float32),
        scratch_types=[
            pltpu.VMEM((idxrows_per_w, 128), jnp.int32),
            pltpu.VMEM((buf_rows, d), jnp.float32),
            pltpu.SemaphoreType.DMA,
        ],
    )
    def k(table_hbm, idx_hbm, out_hbm, idx_v, rows_v, sem):
        wid = lax.axis_index("s") * nc + lax.axis_index("c")
        pltpu.sync_copy(idx_hbm.at[wid], idx_v)
        row_base = wid * rows_per_w

        def do_group(g, carry):
            handles = []
            for j in range(group):
                handles.append(pltpu.async_copy(
                    table_hbm.at[idx_v.at[g * group + j]],
                    rows_v.at[pl.ds(j * 128, 128)],
                    sem,
                ))
            for h in handles:
                h.wait()
            pltpu.sync_copy(rows_v,
                            out_hbm.at[pl.ds(row_base + g * buf_rows,
                                             buf_rows)])
            return carry

        lax.fori_loop(0, n_groups, do_group, 0)

    return k(table, idx2d)


def _tc_dense(w_emb3, w_mask, mask_table, Wh0, bh0, Wz0, bz0, Wh1, bh1,
              Wz1, bz1, W_att, b_att, W_emb, b_emb, W_out, b_out):
    B, L, D = w_emb3.shape
    C = W_out.shape[1]
    BB = 128
    grid = (B // BB,)

    def body(x_ref, m_ref, mt_ref, wh0, bh0_, wz0, bz0_, wh1, bh1_, wz1,
             bz1_, watt, batt, wemb, bemb, wout, bout, out_ref):
        gate = jax.nn.sigmoid(mt_ref[...])
        t = m_ref[...]
        x3 = x_ref[...]
        x = x3.reshape(BB * L, D)
        x = x * (gate[0] + t * (gate[1] - gate[0]))
        for wh, bh, wz, bz in ((wh0, bh0_, wz0, bz0_),
                               (wh1, bh1_, wz1, bz1_)):
            h = jnp.tanh(
                jnp.dot(x, wh[...], preferred_element_type=jnp.float32)
                + bh[...])
            z = jax.nn.sigmoid(
                jnp.dot(x, wz[...], preferred_element_type=jnp.float32)
                + bz[...])
            x = h * z + x * (1.0 - z)
        att = jax.nn.sigmoid(
            jnp.dot(x, watt[...], preferred_element_type=jnp.float32)
            + batt[...])
        emb = jnp.tanh(
            jnp.dot(x, wemb[...], preferred_element_type=jnp.float32)
            + bemb[...])
        g = (att * emb).reshape(BB, L, D)
        pooled = jnp.mean(g, axis=1) + jnp.max(g, axis=1)
        out_ref[...] = (
            jnp.dot(pooled, wout[...], preferred_element_type=jnp.float32)
            + bout[...])

    full = lambda shape: pl.BlockSpec(shape, lambda i: (0,) * len(shape))
    return pl.pallas_call(
        body,
        grid=grid,
        in_specs=[
            pl.BlockSpec((BB, L, D), lambda i: (i, 0, 0)),
            pl.BlockSpec((BB * L, 1), lambda i: (i, 0)),
            full((2, D)),
            full((D, D)), full((1, D)), full((D, D)), full((1, D)),
            full((D, D)), full((1, D)), full((D, D)), full((1, D)),
            full((D, D)), full((1, D)), full((D, D)), full((1, D)),
            full((D, C)), full((1, C)),
        ],
        out_specs=pl.BlockSpec((BB, C), lambda i: (i, 0)),
        out_shape=jax.ShapeDtypeStruct((B, C), jnp.float32),
    )(w_emb3, w_mask, mask_table,
      Wh0, bh0.reshape(1, D), Wz0, bz0.reshape(1, D),
      Wh1, bh1.reshape(1, D), Wz1, bz1.reshape(1, D),
      W_att, b_att.reshape(1, D), W_emb, b_emb.reshape(1, D),
      W_out, b_out.reshape(1, C))


def kernel(words2ids, w_mask, paris_mat, w_table, mask_table, Wh0, bh0,
           Wz0, bz0, Wh1, bh1, Wz1, bz1, W_att, b_att, W_emb, b_emb,
           W_out, b_out):
    B, L = words2ids.shape
    D = w_table.shape[1]
    n = B * L
    idx2d = words2ids.astype(jnp.int32).reshape(32, n // (32 * 128), 128)
    gathered = _sc_gather(w_table, idx2d, n, D)
    w_emb3 = gathered.reshape(B, L, D)
    msel = (w_mask >= 1).astype(jnp.float32).reshape(n, 1)
    return _tc_dense(w_emb3, msel, mask_table,
                     Wh0, bh0, Wz0, bz0, Wh1, bh1, Wz1, bz1,
                     W_att, b_att, W_emb, b_emb, W_out, b_out)


# speedup vs baseline: 1.6925x; 1.1947x over previous
import functools

import jax
import jax.numpy as jnp
from jax import lax
from jax.experimental import pallas as pl
from jax.experimental.pallas import tpu as pltpu
from jax.experimental.pallas import tpu_sc as plsc


def _sc_gather(table, idx2d, n_rows, d):
    info = plsc.get_sparse_core_info()
    nc, ns = info.num_cores, info.num_subcores
    nw = nc * ns
    rows_per_w = n_rows // nw
    idxrows_per_w = rows_per_w // 128
    group = 10
    n_groups = idxrows_per_w // group
    buf_rows = group * 128

    mesh = plsc.VectorSubcoreMesh(core_axis_name="c", subcore_axis_name="s")

    @functools.partial(
        pl.kernel,
        mesh=mesh,
        compiler_params=pltpu.CompilerParams(use_tc_tiling_on_sc=False),
        out_type=jax.ShapeDtypeStruct((n_rows, d), jnp.float32),
        scratch_types=[
            pltpu.VMEM((idxrows_per_w, 128), jnp.int32),
            pltpu.VMEM((buf_rows, d), jnp.float32),
            pltpu.SemaphoreType.DMA,
        ],
    )
    def k(table_hbm, idx_hbm, out_hbm, idx_v, rows_v, sem):
        wid = lax.axis_index("s") * nc + lax.axis_index("c")
        pltpu.sync_copy(idx_hbm.at[wid], idx_v)
        row_base = wid * rows_per_w

        def do_group(g, carry):
            handles = []
            for j in range(group):
                handles.append(pltpu.async_copy(
                    table_hbm.at[idx_v.at[g * group + j]],
                    rows_v.at[pl.ds(j * 128, 128)],
                    sem,
                ))
            for h in handles:
                h.wait()
            pltpu.sync_copy(rows_v,
                            out_hbm.at[pl.ds(row_base + g * buf_rows,
                                             buf_rows)])
            return carry

        lax.fori_loop(0, n_groups, do_group, 0)

    return k(table, idx2d)


def _tc_dense(xp, te, to, mask_table, Wb0, bh20, bz20, Wb1, bh21, bz21,
              Wae, batt2, bemb2, W_out, b_out, B, L, D, C):
    BB = 128
    G = BB * L // 128
    P = BB * L // 2
    PD = L // 2
    grid = (B // BB,)
    f32 = jnp.float32

    def body(x_ref, te_ref, to_ref, mt_ref, wb0, bh0_, bz0_, wb1, bh1_,
             bz1_, wae, batt_, bemb_, wout, bout, out_ref):
        gate = jax.nn.sigmoid(mt_ref[...])
        g0 = jnp.concatenate([gate[0:1], gate[0:1]], axis=1)
        g10 = jnp.concatenate([gate[1:2] - gate[0:1]] * 2, axis=1)
        tp = jnp.concatenate(
            [jnp.broadcast_to(te_ref[...], (P, D)),
             jnp.broadcast_to(to_ref[...], (P, D))], axis=1)
        x = x_ref[...].reshape(P, 2 * D)
        x = x * (g0 + tp * g10)
        for wb, bh, bz in ((wb0, bh0_, bz0_), (wb1, bh1_, bz1_)):
            hz = jnp.dot(x.astype(jnp.bfloat16), wb[...],
                         preferred_element_type=f32)
            h = jnp.tanh(jnp.concatenate(
                [hz[:, 0:D], hz[:, 2 * D:3 * D]], axis=1) + bh[...])
            z = jax.nn.sigmoid(jnp.concatenate(
                [hz[:, D:2 * D], hz[:, 3 * D:4 * D]], axis=1) + bz[...])
            x = h * z + x * (1.0 - z)
        ae = jnp.dot(x.astype(jnp.bfloat16), wae[...],
                     preferred_element_type=f32)
        att = jax.nn.sigmoid(jnp.concatenate(
            [ae[:, 0:D], ae[:, 2 * D:3 * D]], axis=1) + batt_[...])
        emb = jnp.tanh(jnp.concatenate(
            [ae[:, D:2 * D], ae[:, 3 * D:4 * D]], axis=1) + bemb_[...])
        g = (att * emb).reshape(BB, PD, 2 * D)
        s = jnp.sum(g, axis=1)
        mx = jnp.max(g, axis=1)
        pooled = ((s[:, 0:D] + s[:, D:2 * D]) * (1.0 / L)
                  + jnp.maximum(mx[:, 0:D], mx[:, D:2 * D]))
        out_ref[...] = (
            jnp.dot(pooled, wout[...], preferred_element_type=f32)
            + bout[...])

    full = lambda shape: pl.BlockSpec(shape, lambda i: (0,) * len(shape))
    return pl.pallas_call(
        body,
        grid=grid,
        in_specs=[
            pl.BlockSpec((G, D, 2 * D), lambda i: (i, 0, 0)),
            pl.BlockSpec((P, 1), lambda i: (i, 0)),
            pl.BlockSpec((P, 1), lambda i: (i, 0)),
            full((2, D)),
            full((2 * D, 4 * D)), full((1, 2 * D)), full((1, 2 * D)),
            full((2 * D, 4 * D)), full((1, 2 * D)), full((1, 2 * D)),
            full((2 * D, 4 * D)), full((1, 2 * D)), full((1, 2 * D)),
            full((D, C)), full((1, C)),
        ],
        out_specs=pl.BlockSpec((BB, C), lambda i: (i, 0)),
        out_shape=jax.ShapeDtypeStruct((B, C), jnp.float32),
    )(xp, te, to, mask_table, Wb0, bh20, bz20, Wb1, bh21, bz21,
      Wae, batt2, bemb2, W_out, b_out)


def _blockdiag2(w):
    d, k = w.shape
    z = jnp.zeros((d, k), jnp.float32)
    return jnp.block([[w, z], [z, w]]).astype(jnp.bfloat16)


def kernel(words2ids, w_mask, paris_mat, w_table, mask_table, Wh0, bh0,
           Wz0, bz0, Wh1, bh1, Wz1, bz1, W_att, b_att, W_emb, b_emb,
           W_out, b_out):
    B, L = words2ids.shape
    D = w_table.shape[1]
    C = W_out.shape[1]
    n = B * L
    idx2d = words2ids.astype(jnp.int32).reshape(32, n // (32 * 128), 128)
    gathered = _sc_gather(w_table, idx2d, n, D)
    xp = gathered.reshape(n // 128, D, 128)
    msel = (w_mask >= 1).astype(jnp.float32).reshape(n)
    te = msel[0::2].reshape(n // 2, 1)
    to = msel[1::2].reshape(n // 2, 1)
    dup = lambda v: jnp.concatenate([v, v]).reshape(1, 2 * D)
    return _tc_dense(
        xp, te, to, mask_table,
        _blockdiag2(jnp.concatenate([Wh0, Wz0], axis=1)), dup(bh0), dup(bz0),
        _blockdiag2(jnp.concatenate([Wh1, Wz1], axis=1)), dup(bh1), dup(bz1),
        _blockdiag2(jnp.concatenate([W_att, W_emb], axis=1)),
        dup(b_att), dup(b_emb),
        W_out, b_out.reshape(1, C), B, L, D, C)


# speedup vs baseline: 1.7727x; 1.0474x over previous
import functools

import jax
import jax.numpy as jnp
from jax import lax
from jax.experimental import pallas as pl
from jax.experimental.pallas import tpu as pltpu
from jax.experimental.pallas import tpu_sc as plsc


def _sc_gather(table, idx3d, n_rows, d):
    info = plsc.get_sparse_core_info()
    nc, ns = info.num_cores, info.num_subcores
    nw = nc * ns
    rows_per_w = n_rows // nw
    idxrows_per_w = rows_per_w // 128
    group = 5
    n_groups = idxrows_per_w // group
    buf_rows = group * 128

    mesh = plsc.VectorSubcoreMesh(core_axis_name="c", subcore_axis_name="s")

    @functools.partial(
        pl.kernel,
        mesh=mesh,
        compiler_params=pltpu.CompilerParams(use_tc_tiling_on_sc=False),
        out_type=jax.ShapeDtypeStruct((n_rows, d), jnp.float32),
        scratch_types=[
            pltpu.VMEM((idxrows_per_w, 128), jnp.int32),
            pltpu.VMEM((buf_rows, d), jnp.float32),
            pltpu.SemaphoreType.DMA,
        ],
    )
    def k(table_hbm, idx_hbm, out_hbm, idx_v, rows_v, sem):
        wid = lax.axis_index("s") * nc + lax.axis_index("c")
        pltpu.sync_copy(idx_hbm.at[wid], idx_v)
        row_base = wid * rows_per_w

        def do_group(g, carry):
            handles = []
            for j in range(group):
                handles.append(pltpu.async_copy(
                    table_hbm.at[idx_v.at[g * group + j]],
                    rows_v.at[pl.ds(j * 128, 128)],
                    sem,
                ))
            for h in handles:
                h.wait()
            pltpu.sync_copy(rows_v,
                            out_hbm.at[pl.ds(row_base + g * buf_rows,
                                             buf_rows)])
            return carry

        lax.fori_loop(0, n_groups, do_group, 0)

    return k(table, idx3d)


def _tc_dense(xg, t, mask_table, Wb0, bh0p, bz0p, Wb1, bh1p, bz1p,
              Wae, battp, bembp, Woutp, boutp, B, L, D, C):
    BB = 128
    P = BB * L
    grid = (B // BB,)
    f32 = jnp.float32

    def body(x_ref, t_ref, mt_ref, wb0, bh0_, bz0_, wb1, bh1_, bz1_,
             wae, batt_, bemb_, wout, bout, out_ref):
        gate = jax.nn.sigmoid(mt_ref[...])
        g0 = jnp.concatenate([gate[0:1]] * 2, axis=1)
        g10 = jnp.concatenate([gate[1:2] - gate[0:1]] * 2, axis=1)
        x = x_ref[...] * (g0 + t_ref[...] * g10)
        for wb, bh, bz in ((wb0, bh0_, bz0_), (wb1, bh1_, bz1_)):
            hz = jnp.dot(x.astype(jnp.bfloat16), wb[...],
                         preferred_element_type=f32)
            h = jnp.tanh(hz[:, 0:2 * D] + bh[...])
            z = jax.nn.sigmoid(hz[:, 2 * D:4 * D] + bz[...])
            x = h * z + x * (1.0 - z)
        ae = jnp.dot(x.astype(jnp.bfloat16), wae[...],
                     preferred_element_type=f32)
        att = jax.nn.sigmoid(ae[:, 0:2 * D] + batt_[...])
        emb = jnp.tanh(ae[:, 2 * D:4 * D] + bemb_[...])
        g = (att * emb).reshape(BB, L, 2 * D)
        pooled = (jnp.sum(g, axis=1) * (1.0 / L)
                  + jnp.max(g, axis=1))
        out_ref[...] = (
            jnp.dot(pooled, wout[...], preferred_element_type=f32)
            + bout[...])

    full = lambda shape: pl.BlockSpec(shape, lambda i: (0,) * len(shape))
    return pl.pallas_call(
        body,
        grid=grid,
        in_specs=[
            pl.BlockSpec((P, 2 * D), lambda i: (i, 0)),
            pl.BlockSpec((P, 1), lambda i: (i, 0)),
            full((2, D)),
            full((2 * D, 4 * D)), full((1, 2 * D)), full((1, 2 * D)),
            full((2 * D, 4 * D)), full((1, 2 * D)), full((1, 2 * D)),
            full((2 * D, 4 * D)), full((1, 2 * D)), full((1, 2 * D)),
            full((2 * D, C)), full((1, C)),
        ],
        out_specs=pl.BlockSpec((BB, C), lambda i: (i, 0)),
        out_shape=jax.ShapeDtypeStruct((B, C), jnp.float32),
    )(xg, t, mask_table, Wb0, bh0p, bz0p, Wb1, bh1p, bz1p,
      Wae, battp, bembp, Woutp, boutp)


def kernel(words2ids, w_mask, paris_mat, w_table, mask_table, Wh0, bh0,
           Wz0, bz0, Wh1, bh1, Wz1, bz1, W_att, b_att, W_emb, b_emb,
           W_out, b_out):
    B, L = words2ids.shape
    V, D = w_table.shape
    C = W_out.shape[1]
    n = B * L
    wpad = jnp.pad(w_table, ((0, 0), (0, D)))
    idx3d = words2ids.astype(jnp.int32).reshape(32, n // (32 * 128), 128)
    gathered = _sc_gather(wpad, idx3d, n, 2 * D)
    msel = (w_mask >= 1).astype(jnp.float32).reshape(n, 1)
    z = jnp.zeros((D, D), jnp.float32)
    zr = jnp.zeros((D, 4 * D), jnp.float32)

    def wide(wa, wb):
        top = jnp.concatenate([wa, z, wb, z], axis=1)
        return jnp.concatenate([top, zr], axis=0).astype(jnp.bfloat16)

    pad1 = lambda v: jnp.concatenate([v, jnp.zeros((D,), jnp.float32)]
                                     ).reshape(1, 2 * D)
    return _tc_dense(
        gathered, msel, mask_table,
        wide(Wh0, Wz0), pad1(bh0), pad1(bz0),
        wide(Wh1, Wz1), pad1(bh1), pad1(bz1),
        wide(W_att, W_emb), pad1(b_att), pad1(b_emb),
        jnp.concatenate([W_out, jnp.zeros((D, C), jnp.float32)], axis=0),
        b_out.reshape(1, C), B, L, D, C)


# speedup vs baseline: 2.2856x; 1.2893x over previous
import functools

import jax
import jax.numpy as jnp
from jax import lax
from jax.experimental import pallas as pl
from jax.experimental.pallas import tpu as pltpu
from jax.experimental.pallas import tpu_sc as plsc


def _sc_gather(table, idx3d, n_rows, d):
    info = plsc.get_sparse_core_info()
    nc, ns = info.num_cores, info.num_subcores
    nw = nc * ns
    rows_per_w = n_rows // nw
    idxrows_per_w = rows_per_w // 128
    group = 5
    n_groups = idxrows_per_w // group
    buf_rows = group * 128

    mesh = plsc.VectorSubcoreMesh(core_axis_name="c", subcore_axis_name="s")

    @functools.partial(
        pl.kernel,
        mesh=mesh,
        compiler_params=pltpu.CompilerParams(use_tc_tiling_on_sc=False),
        out_type=jax.ShapeDtypeStruct((n_rows, d), jnp.float32),
        scratch_types=[
            pltpu.VMEM((idxrows_per_w, 128), jnp.int32),
            pltpu.VMEM((buf_rows, d), jnp.float32),
            pltpu.SemaphoreType.DMA,
        ],
    )
    def k(table_hbm, idx_hbm, out_hbm, idx_v, rows_v, sem):
        wid = lax.axis_index("s") * nc + lax.axis_index("c")
        pltpu.sync_copy(idx_hbm.at[wid], idx_v)
        row_base = wid * rows_per_w

        def do_group(g, carry):
            handles = []
            for j in range(group):
                handles.append(pltpu.async_copy(
                    table_hbm.at[idx_v.at[g * group + j]],
                    rows_v.at[pl.ds(j * 128, 128)],
                    sem,
                ))
            for h in handles:
                h.wait()
            pltpu.sync_copy(rows_v,
                            out_hbm.at[pl.ds(row_base + g * buf_rows,
                                             buf_rows)])
            return carry

        lax.fori_loop(0, n_groups, do_group, 0)

    return k(table, idx3d)


def _tc_transpose_pad(wT, V, D):
    BLK = 4096
    grid = (pl.cdiv(V, BLK),)

    def body(x_ref, out_ref):
        out_ref[:, 0:D] = x_ref[...].T
        out_ref[:, D:2 * D] = jnp.zeros((BLK, D), jnp.float32)

    return pl.pallas_call(
        body,
        grid=grid,
        in_specs=[pl.BlockSpec((D, BLK), lambda i: (0, i))],
        out_specs=pl.BlockSpec((BLK, 2 * D), lambda i: (i, 0)),
        out_shape=jax.ShapeDtypeStruct((V, 2 * D), jnp.float32),
    )(wT)


def _tc_dense(xg, t, mask_table, Wb0, bh0p, bz0p, Wb1, bh1p, bz1p,
              Wae, battp, bembp, Woutp, boutp, B, L, D, C):
    BB = 256
    P = BB * L
    grid = (B // BB,)
    f32 = jnp.float32

    def body(x_ref, t_ref, mt_ref, wb0, bh0_, bz0_, wb1, bh1_,
             bz1_, wae, batt_, bemb_, wout, bout, out_ref):
        gate = jax.nn.sigmoid(mt_ref[...])
        g0 = jnp.concatenate([gate[0:1]] * 2, axis=1)
        g10 = jnp.concatenate([gate[1:2] - gate[0:1]] * 2, axis=1)
        lane = lax.broadcasted_iota(jnp.int32, (1, 2 * D), 1)
        x = jnp.where(lane < D,
                      x_ref[...] * (g0 + t_ref[...] * g10),
                      0.0)
        for wb, bh, bz in ((wb0, bh0_, bz0_), (wb1, bh1_, bz1_)):
            hz = jnp.dot(x.astype(jnp.bfloat16), wb[...],
                         preferred_element_type=f32)
            h = jnp.tanh(hz[:, 0:2 * D] + bh[...])
            z = jax.nn.sigmoid(hz[:, 2 * D:4 * D] + bz[...])
            x = h * z + x * (1.0 - z)
        ae = jnp.dot(x.astype(jnp.bfloat16), wae[...],
                     preferred_element_type=f32)
        att = jax.nn.sigmoid(ae[:, 0:2 * D] + batt_[...])
        emb = jnp.tanh(ae[:, 2 * D:4 * D] + bemb_[...])
        g = (att * emb).reshape(BB, L, 2 * D)
        pooled = (jnp.sum(g, axis=1) * (1.0 / L)
                  + jnp.max(g, axis=1))
        out_ref[...] = (
            jnp.dot(pooled, wout[...], preferred_element_type=f32)
            + bout[...])

    full = lambda shape: pl.BlockSpec(shape, lambda i: (0,) * len(shape))
    return pl.pallas_call(
        body,
        grid=grid,
        in_specs=[
            pl.BlockSpec((P, 2 * D), lambda i: (i, 0)),
            pl.BlockSpec((P, 1), lambda i: (i, 0)),
            full((2, D)),
            full((2 * D, 4 * D)), full((1, 2 * D)), full((1, 2 * D)),
            full((2 * D, 4 * D)), full((1, 2 * D)), full((1, 2 * D)),
            full((2 * D, 4 * D)), full((1, 2 * D)), full((1, 2 * D)),
            full((2 * D, C)), full((1, C)),
        ],
        out_specs=pl.BlockSpec((BB, C), lambda i: (i, 0)),
        out_shape=jax.ShapeDtypeStruct((B, C), jnp.float32),
    )(xg, t, mask_table, Wb0, bh0p, bz0p, Wb1, bh1p, bz1p,
      Wae, battp, bembp, Woutp, boutp)


def kernel(words2ids, w_mask, paris_mat, w_table, mask_table, Wh0, bh0,
           Wz0, bz0, Wh1, bh1, Wz1, bz1, W_att, b_att, W_emb, b_emb,
           W_out, b_out):
    B, L = words2ids.shape
    V, D = w_table.shape
    C = W_out.shape[1]
    n = B * L
    wpad = _tc_transpose_pad(w_table.T, V, D)
    idx = words2ids.astype(jnp.int32).reshape(n)
    msel = (w_mask >= 1).astype(jnp.float32).reshape(n, 1)
    z = jnp.zeros((D, D), jnp.float32)
    zr = jnp.zeros((D, 4 * D), jnp.float32)

    def wide(wa, wb):
        top = jnp.concatenate([wa, z, wb, z], axis=1)
        return jnp.concatenate([top, zr], axis=0).astype(jnp.bfloat16)

    pad1 = lambda v: jnp.concatenate([v, jnp.zeros((D,), jnp.float32)]
                                     ).reshape(1, 2 * D)
    wargs = (wide(Wh0, Wz0), pad1(bh0), pad1(bz0),
             wide(Wh1, Wz1), pad1(bh1), pad1(bz1),
             wide(W_att, W_emb), pad1(b_att), pad1(b_emb),
             jnp.concatenate([W_out, jnp.zeros((D, C), jnp.float32)],
                             axis=0),
             b_out.reshape(1, C))
    idx3d = idx.reshape(32, n // (32 * 128), 128)
    gath = _sc_gather(wpad, idx3d, n, 2 * D)
    return _tc_dense(gath, msel, mask_table, *wargs, B, L, D, C)


# speedup vs baseline: 2.5731x; 1.1258x over previous
import functools

import jax
import jax.numpy as jnp
from jax import lax
from jax.experimental import pallas as pl
from jax.experimental.pallas import tpu as pltpu
from jax.experimental.pallas import tpu_sc as plsc


def _sc_gather(table, idx3d, n_rows, d):
    info = plsc.get_sparse_core_info()
    nc, ns = info.num_cores, info.num_subcores
    nw = nc * ns
    rows_per_w = n_rows // nw
    idxrows_per_w = rows_per_w // 128
    group = 5
    n_groups = idxrows_per_w // group
    buf_rows = group * 128

    mesh = plsc.VectorSubcoreMesh(core_axis_name="c", subcore_axis_name="s")

    @functools.partial(
        pl.kernel,
        mesh=mesh,
        compiler_params=pltpu.CompilerParams(use_tc_tiling_on_sc=False),
        out_type=jax.ShapeDtypeStruct((n_rows, d), jnp.float32),
        scratch_types=[
            pltpu.VMEM((idxrows_per_w, 128), jnp.int32),
            pltpu.VMEM((buf_rows, d), jnp.float32),
            pltpu.SemaphoreType.DMA,
        ],
    )
    def k(table_hbm, idx_hbm, out_hbm, idx_v, rows_v, sem):
        wid = lax.axis_index("s") * nc + lax.axis_index("c")
        pltpu.sync_copy(idx_hbm.at[wid], idx_v)
        row_base = wid * rows_per_w

        def do_group(g, carry):
            handles = []
            for j in range(group):
                handles.append(pltpu.async_copy(
                    table_hbm.at[idx_v.at[g * group + j]],
                    rows_v.at[pl.ds(j * 128, 128)],
                    sem,
                ))
            for h in handles:
                h.wait()
            pltpu.sync_copy(rows_v,
                            out_hbm.at[pl.ds(row_base + g * buf_rows,
                                             buf_rows)])
            return carry

        lax.fori_loop(0, n_groups, do_group, 0)

    return k(table, idx3d)


def _tc_transpose_pad(wT, V, D):
    BLK = 8192
    grid = (pl.cdiv(V, BLK),)

    def body(x_ref, out_ref):
        out_ref[:, 0:D] = x_ref[...].T
        out_ref[:, D:2 * D] = jnp.zeros((BLK, D), jnp.float32)

    return pl.pallas_call(
        body,
        grid=grid,
        in_specs=[pl.BlockSpec((D, BLK), lambda i: (0, i))],
        out_specs=pl.BlockSpec((BLK, 2 * D), lambda i: (i, 0)),
        out_shape=jax.ShapeDtypeStruct((V, 2 * D), jnp.float32),
    )(wT)


def _tc_dense(xg, t, mask_table, Wb0, bh0p, bz0p, Wb1, bh1p, bz1p,
              Wae, battp, bembp, Woutp, boutp, B, L, D, C):
    BB = 256
    P = BB * L
    grid = (B // BB,)
    f32 = jnp.float32

    def body(x_ref, t_ref, mt_ref, wb0, bh0_, bz0_, wb1, bh1_,
             bz1_, wae, batt_, bemb_, wout, bout, out_ref):
        gate = jax.nn.sigmoid(mt_ref[...])
        g0 = jnp.concatenate([gate[0:1]] * 2, axis=1)
        g10 = jnp.concatenate([gate[1:2] - gate[0:1]] * 2, axis=1)
        lane = lax.broadcasted_iota(jnp.int32, (1, 2 * D), 1)
        x = jnp.where(lane < D,
                      x_ref[...] * (g0 + t_ref[...] * g10),
                      0.0)
        for wb, bh, bz in ((wb0, bh0_, bz0_), (wb1, bh1_, bz1_)):
            hz = jnp.dot(x.astype(jnp.bfloat16), wb[...],
                         preferred_element_type=f32)
            h = jnp.tanh(hz[:, 0:2 * D] + bh[...])
            z = jax.nn.sigmoid(hz[:, 2 * D:4 * D] + bz[...])
            x = h * z + x * (1.0 - z)
        ae = jnp.dot(x.astype(jnp.bfloat16), wae[...],
                     preferred_element_type=f32)
        att = jax.nn.sigmoid(ae[:, 0:2 * D] + batt_[...])
        emb = jnp.tanh(ae[:, 2 * D:4 * D] + bemb_[...])
        g = (att * emb).reshape(BB, L, 2 * D)
        pooled = (jnp.sum(g, axis=1) * (1.0 / L)
                  + jnp.max(g, axis=1))
        out_ref[...] = (
            jnp.dot(pooled, wout[...], preferred_element_type=f32)
            + bout[...])

    full = lambda shape: pl.BlockSpec(shape, lambda i: (0,) * len(shape))
    return pl.pallas_call(
        body,
        grid=grid,
        in_specs=[
            pl.BlockSpec((P, 2 * D), lambda i: (i, 0)),
            pl.BlockSpec((P, 1), lambda i: (i, 0)),
            full((2, D)),
            full((2 * D, 4 * D)), full((1, 2 * D)), full((1, 2 * D)),
            full((2 * D, 4 * D)), full((1, 2 * D)), full((1, 2 * D)),
            full((2 * D, 4 * D)), full((1, 2 * D)), full((1, 2 * D)),
            full((2 * D, C)), full((1, C)),
        ],
        out_specs=pl.BlockSpec((BB, C), lambda i: (i, 0)),
        out_shape=jax.ShapeDtypeStruct((B, C), jnp.float32),
    )(xg, t, mask_table, Wb0, bh0p, bz0p, Wb1, bh1p, bz1p,
      Wae, battp, bembp, Woutp, boutp)


def kernel(words2ids, w_mask, paris_mat, w_table, mask_table, Wh0, bh0,
           Wz0, bz0, Wh1, bh1, Wz1, bz1, W_att, b_att, W_emb, b_emb,
           W_out, b_out):
    B, L = words2ids.shape
    V, D = w_table.shape
    C = W_out.shape[1]
    n = B * L
    wpad = _tc_transpose_pad(w_table.T, V, D)
    idx = words2ids.astype(jnp.int32).reshape(n)
    msel = (w_mask >= 1).astype(jnp.float32).reshape(n, 1)
    z = jnp.zeros((D, D), jnp.float32)
    zr = jnp.zeros((D, 4 * D), jnp.float32)

    def wide(wa, wb):
        top = jnp.concatenate([wa, z, wb, z], axis=1)
        return jnp.concatenate([top, zr], axis=0).astype(jnp.bfloat16)

    pad1 = lambda v: jnp.concatenate([v, jnp.zeros((D,), jnp.float32)]
                                     ).reshape(1, 2 * D)
    wargs = (wide(Wh0, Wz0), pad1(bh0), pad1(bz0),
             wide(Wh1, Wz1), pad1(bh1), pad1(bz1),
             wide(W_att, W_emb), pad1(b_att), pad1(b_emb),
             jnp.concatenate([W_out, jnp.zeros((D, C), jnp.float32)],
                             axis=0),
             b_out.reshape(1, C))
    idx3d = idx.reshape(32, n // (32 * 128), 128)
    gath = _sc_gather(wpad, idx3d, n, 2 * D)
    return _tc_dense(gath, msel, mask_table, *wargs, B, L, D, C)


# speedup vs baseline: 2.6565x; 1.0324x over previous
import functools

import jax
import jax.numpy as jnp
from jax import lax
from jax.experimental import pallas as pl
from jax.experimental.pallas import tpu as pltpu
from jax.experimental.pallas import tpu_sc as plsc


def _sc_gather(table, idx3d, n_rows, d):
    info = plsc.get_sparse_core_info()
    nc, ns = info.num_cores, info.num_subcores
    nw = nc * ns
    rows_per_w = n_rows // nw
    idxrows_per_w = rows_per_w // 128
    group = 5
    n_groups = idxrows_per_w // group
    buf_rows = group * 128

    mesh = plsc.VectorSubcoreMesh(core_axis_name="c", subcore_axis_name="s")

    @functools.partial(
        pl.kernel,
        mesh=mesh,
        compiler_params=pltpu.CompilerParams(use_tc_tiling_on_sc=False),
        out_type=jax.ShapeDtypeStruct((n_rows, d), jnp.float32),
        scratch_types=[
            pltpu.VMEM((idxrows_per_w, 128), jnp.int32),
            pltpu.VMEM((buf_rows, d), jnp.float32),
            pltpu.SemaphoreType.DMA,
        ],
    )
    def k(table_hbm, idx_hbm, out_hbm, idx_v, rows_v, sem):
        wid = lax.axis_index("s") * nc + lax.axis_index("c")
        pltpu.sync_copy(idx_hbm.at[wid], idx_v)
        row_base = wid * rows_per_w

        def do_group(g, carry):
            handles = []
            for j in range(group):
                handles.append(pltpu.async_copy(
                    table_hbm.at[idx_v.at[g * group + j]],
                    rows_v.at[pl.ds(j * 128, 128)],
                    sem,
                ))
            for h in handles:
                h.wait()
            pltpu.sync_copy(rows_v,
                            out_hbm.at[pl.ds(row_base + g * buf_rows,
                                             buf_rows)])
            return carry

        lax.fori_loop(0, n_groups, do_group, 0)

    return k(table, idx3d)


def _tc_transpose_pad(wT, V, D):
    BLK = 16384
    grid = (pl.cdiv(V, BLK),)

    def body(x_ref, out_ref):
        out_ref[:, 0:D] = x_ref[...].T
        out_ref[:, D:2 * D] = jnp.zeros((BLK, D), jnp.float32)

    return pl.pallas_call(
        body,
        grid=grid,
        in_specs=[pl.BlockSpec((D, BLK), lambda i: (0, i))],
        out_specs=pl.BlockSpec((BLK, 2 * D), lambda i: (i, 0)),
        out_shape=jax.ShapeDtypeStruct((V, 2 * D), jnp.float32),
    )(wT)


def _tc_dense(xg, t, mask_table, Wb0, bh0p, bz0p, Wb1, bh1p, bz1p,
              Wae, battp, bembp, Woutp, boutp, B, L, D, C):
    BB = 256
    P = BB * L
    grid = (B // BB,)
    f32 = jnp.float32

    def body(x_ref, t_ref, mt_ref, wb0, bh0_, bz0_, wb1, bh1_,
             bz1_, wae, batt_, bemb_, wout, bout, out_ref):
        gate = jax.nn.sigmoid(mt_ref[...])
        g0 = jnp.concatenate([gate[0:1]] * 2, axis=1)
        g10 = jnp.concatenate([gate[1:2] - gate[0:1]] * 2, axis=1)
        lane = lax.broadcasted_iota(jnp.int32, (1, 2 * D), 1)
        x = jnp.where(lane < D,
                      x_ref[...] * (g0 + t_ref[...] * g10),
                      0.0)
        for wb, bh, bz in ((wb0, bh0_, bz0_), (wb1, bh1_, bz1_)):
            hz = jnp.dot(x.astype(jnp.bfloat16), wb[...],
                         preferred_element_type=f32)
            h = jnp.tanh(hz[:, 0:2 * D] + bh[...])
            z = jax.nn.sigmoid(hz[:, 2 * D:4 * D] + bz[...])
            x = h * z + x * (1.0 - z)
        ae = jnp.dot(x.astype(jnp.bfloat16), wae[...],
                     preferred_element_type=f32)
        att = jax.nn.sigmoid(ae[:, 0:2 * D] + batt_[...])
        emb = jnp.tanh(ae[:, 2 * D:4 * D] + bemb_[...])
        g = (att * emb).reshape(BB, L, 2 * D)
        pooled = (jnp.sum(g, axis=1) * (1.0 / L)
                  + jnp.max(g, axis=1))
        out_ref[...] = (
            jnp.dot(pooled, wout[...], preferred_element_type=f32)
            + bout[...])

    full = lambda shape: pl.BlockSpec(shape, lambda i: (0,) * len(shape))
    return pl.pallas_call(
        body,
        grid=grid,
        in_specs=[
            pl.BlockSpec((P, 2 * D), lambda i: (i, 0)),
            pl.BlockSpec((P, 1), lambda i: (i, 0)),
            full((2, D)),
            full((2 * D, 4 * D)), full((1, 2 * D)), full((1, 2 * D)),
            full((2 * D, 4 * D)), full((1, 2 * D)), full((1, 2 * D)),
            full((2 * D, 4 * D)), full((1, 2 * D)), full((1, 2 * D)),
            full((2 * D, C)), full((1, C)),
        ],
        out_specs=pl.BlockSpec((BB, C), lambda i: (i, 0)),
        out_shape=jax.ShapeDtypeStruct((B, C), jnp.float32),
    )(xg, t, mask_table, Wb0, bh0p, bz0p, Wb1, bh1p, bz1p,
      Wae, battp, bembp, Woutp, boutp)


def kernel(words2ids, w_mask, paris_mat, w_table, mask_table, Wh0, bh0,
           Wz0, bz0, Wh1, bh1, Wz1, bz1, W_att, b_att, W_emb, b_emb,
           W_out, b_out):
    B, L = words2ids.shape
    V, D = w_table.shape
    C = W_out.shape[1]
    n = B * L
    wpad = _tc_transpose_pad(w_table.T, V, D)
    idx = words2ids.astype(jnp.int32).reshape(n)
    msel = (w_mask >= 1).astype(jnp.float32).reshape(n, 1)
    z = jnp.zeros((D, D), jnp.float32)
    zr = jnp.zeros((D, 4 * D), jnp.float32)

    def wide(wa, wb):
        top = jnp.concatenate([wa, z, wb, z], axis=1)
        return jnp.concatenate([top, zr], axis=0).astype(jnp.bfloat16)

    pad1 = lambda v: jnp.concatenate([v, jnp.zeros((D,), jnp.float32)]
                                     ).reshape(1, 2 * D)
    wargs = (wide(Wh0, Wz0), pad1(bh0), pad1(bz0),
             wide(Wh1, Wz1), pad1(bh1), pad1(bz1),
             wide(W_att, W_emb), pad1(b_att), pad1(b_emb),
             jnp.concatenate([W_out, jnp.zeros((D, C), jnp.float32)],
                             axis=0),
             b_out.reshape(1, C))
    idx3d = idx.reshape(32, n // (32 * 128), 128)
    gath = _sc_gather(wpad, idx3d, n, 2 * D)
    return _tc_dense(gath, msel, mask_table, *wargs, B, L, D, C)
